# Initial kernel scaffold; baseline (speedup 1.0000x reference)
import jax
import jax.numpy as jnp
from jax.experimental import pallas as pl


def kernel(captions, caption_lengths, logs, idall, dfall, ix, emb_table, W_out, b_out):
    raise NotImplementedError("write your pallas kernel here")


# speedup vs baseline: 25.6294x; 25.6294x over previous
import functools

import jax
import jax.numpy as jnp
from jax import lax
from jax.experimental import pallas as pl
from jax.experimental.pallas import tpu as pltpu
from jax.experimental.pallas import tpu_sc as plsc

_NC, _NS, _L = 2, 16, 16
_NW = _NC * _NS


def _proj_body(emb_ref, w_ref, t_ref):
    t_ref[...] = jnp.sum(emb_ref[...] * w_ref[...], axis=-1)


def _project_table(emb_table, w_row):
    V, D = emb_table.shape
    G, rb = 20, 625
    emb4 = emb_table.reshape(G, rb, 8, D)
    w4 = w_row.reshape(1, 1, 1, D)
    t3 = pl.pallas_call(
        _proj_body,
        grid=(G,),
        in_specs=[
            pl.BlockSpec((1, rb, 8, D), lambda i: (i, 0, 0, 0)),
            pl.BlockSpec((1, 1, 1, D), lambda i: (0, 0, 0, 0)),
        ],
        out_specs=pl.BlockSpec((1, rb, 8), lambda i: (i, 0, 0)),
        out_shape=jax.ShapeDtypeStruct((G, rb, 8), jnp.float32),
    )(emb4, w4)
    return t3.reshape(V)


def _make_sc_sum(V, S, rpt):
    nchunk = rpt // _L
    mesh = plsc.VectorSubcoreMesh(core_axis_name="c", subcore_axis_name="s")

    @functools.partial(
        pl.kernel,
        mesh=mesh,
        out_type=jax.ShapeDtypeStruct((_NW, rpt), jnp.float32),
        scratch_types=[
            pltpu.VMEM((V,), jnp.float32),
            pltpu.VMEM((S, rpt), jnp.int32),
            pltpu.VMEM((rpt,), jnp.int32),
            pltpu.VMEM((rpt,), jnp.float32),
        ],
        compiler_params=pltpu.CompilerParams(needs_layout_passes=False),
    )
    def sc_sum(t_hbm, cap_hbm, len_hbm, out_hbm, t_v, cap_v, len_v, o_v):
        wid = lax.axis_index("s") * _NC + lax.axis_index("c")
        pltpu.sync_copy(t_hbm, t_v)
        pltpu.sync_copy(cap_hbm.at[wid], cap_v)
        pltpu.sync_copy(len_hbm.at[wid], len_v)

        def body(c, carry):
            base = c * _L
            l16 = len_v[pl.ds(base, _L)]
            acc = jnp.zeros((_L,), jnp.float32)
            for j in range(S):
                idx = cap_v[j, pl.ds(base, _L)]
                idxm = jnp.where(l16 > (j + 1), idx, 0)
                acc = acc + plsc.load_gather(t_v, [idxm])
            o_v[pl.ds(base, _L)] = acc
            return carry

        lax.fori_loop(0, nchunk, body, 0)
        pltpu.sync_copy(o_v, out_hbm.at[wid])

    return sc_sum


def _post_body(s_ref, dfall_ref, idall_ref, ix_ref, b_ref, out_ref):
    idf = dfall_ref[...] * (idall_ref[...] == ix_ref[...]).astype(jnp.float32)
    sc = (s_ref[...] + b_ref[0, 0]) * idf
    m = jnp.max(sc, axis=1, keepdims=True)
    e = jnp.exp(sc - m)
    lse = jnp.log(jnp.sum(e, axis=1, keepdims=True)) + m
    out_ref[...] = sc - lse


def kernel(captions, caption_lengths, logs, idall, dfall, ix, emb_table, W_out, b_out):
    del logs
    B, BEAM, S = captions.shape
    V, D = emb_table.shape
    R = B * BEAM
    rpt = R // _NW

    t = _project_table(emb_table, W_out[0])

    cap_t = captions.reshape(R, S).T.reshape(S, _NW, rpt).transpose(1, 0, 2)
    len_t = caption_lengths.reshape(R).reshape(_NW, rpt)

    sraw = _make_sc_sum(V, S, rpt)(t, cap_t, len_t).reshape(B, BEAM)

    out = pl.pallas_call(
        _post_body,
        out_shape=jax.ShapeDtypeStruct((B, BEAM), jnp.float32),
    )(sraw, dfall, idall, ix.reshape(B, 1), b_out.reshape(1, 1))
    return out
